# baseline (speedup 1.0000x reference)
import functools

import jax
import jax.numpy as jnp
from jax import lax
from jax.experimental import pallas as pl
from jax.experimental.pallas import tpu as pltpu
from jax.experimental.pallas import tpu_sc as plsc

B, L, D = 4096, 200, 16
NT = 4
NC, NS = 2, 16
NW = NC * NS
BPW = B // NW
NL = 4
NCHUNK = L // NL
ROWS = NL * NT * BPW
UB = 8

_mesh = plsc.VectorSubcoreMesh(core_axis_name="c", subcore_axis_name="s")


@functools.partial(
    pl.kernel,
    out_type=jax.ShapeDtypeStruct((L, NT * D, B), jnp.float32),
    mesh=_mesh,
    compiler_params=pltpu.CompilerParams(
        use_tc_tiling_on_sc=False, needs_layout_passes=False
    ),
    scratch_types=[
        pltpu.VMEM((2, NT, NL, BPW), jnp.int32),
        pltpu.VMEM((2 * ROWS, D), jnp.float32),
        pltpu.VMEM((NL, NT * D, BPW), jnp.float32),
        pltpu.SemaphoreType.DMA,
        pltpu.SemaphoreType.DMA,
    ],
)
def _embed_kernel(w0, w1, w2, w3, i0, i1, i2, i3, out_hbm, idx_v, stage_v, outbuf_v, sem0, sem1):
    tables = (w0, w1, w2, w3)
    idxs = (i0, i1, i2, i3)
    sems = (sem0, sem1)
    wid = lax.axis_index("s") * NC + lax.axis_index("c")
    b0 = wid * BPW
    lane = lax.iota(jnp.int32, 16)
    d_idx = [lane + t * D for t in range(NT)]

    def fetch(c, slot):
        l0 = c * NL
        for t in range(NT):
            pltpu.sync_copy(idxs[t].at[pl.ds(l0, NL), pl.ds(b0, BPW)], idx_v.at[slot, t])
        for l in range(NL):
            for t in range(NT):
                pltpu.async_copy(
                    tables[t].at[idx_v.at[slot, t, l]],
                    stage_v.at[pl.ds(slot * ROWS + (l * NT + t) * BPW, BPW)],
                    sems[slot],
                )

    def drain(slot):
        pltpu.make_async_copy(
            w2.at[pl.ds(0, ROWS)], stage_v.at[pl.ds(slot * ROWS, ROWS)], sems[slot]
        ).wait()

    def process(c, slot):
        base = slot * ROWS

        @plsc.parallel_loop(0, BPW, 1, unroll=UB)
        def tbody(bb):
            bvec = jnp.full((16,), 0, jnp.int32) + bb
            vs = [
                stage_v[base + r * BPW + bb, :] for r in range(NL * NT)
            ]
            for l in range(NL):
                lvec = jnp.full((16,), l, jnp.int32)
                for t in range(NT):
                    plsc.store_scatter(
                        outbuf_v, [lvec, d_idx[t], bvec], vs[l * NT + t]
                    )
        l0 = c * NL
        for l in range(NL):
            pltpu.sync_copy(outbuf_v.at[l], out_hbm.at[l0 + l, :, pl.ds(b0, BPW)])

    fetch(0, 0)

    def body(k, carry):
        c0 = 2 * k
        fetch(c0 + 1, 1)
        drain(0)
        process(c0, 0)

        @pl.when(c0 + 2 < NCHUNK)
        def _():
            fetch(c0 + 2, 0)

        drain(1)
        process(c0 + 1, 1)
        return carry

    lax.fori_loop(0, NCHUNK // 2, body, 0)


def kernel(event_name, level, fqid, room_fqid, W_event_name, W_level, W_fqid, W_room_fqid):
    iT = [a.astype(jnp.int32).T for a in (event_name, level, fqid, room_fqid)]
    out = _embed_kernel(W_event_name, W_level, W_fqid, W_room_fqid, *iT)
    return out.transpose(2, 0, 1)


# speedup vs baseline: 2.3371x; 2.3371x over previous
import functools

import jax
import jax.numpy as jnp
from jax import lax
from jax.experimental import pallas as pl
from jax.experimental.pallas import tpu as pltpu
from jax.experimental.pallas import tpu_sc as plsc

B, L, D = 4096, 200, 16
NT = 4
NC, NS = 2, 16
NW = NC * NS
BPW = B // NW
NL = 4
NCHUNK = L // NL
ROWS = NL * NT * BPW
UB = 8
OBP = BPW + 1

_mesh = plsc.VectorSubcoreMesh(core_axis_name="c", subcore_axis_name="s")


@functools.partial(
    pl.kernel,
    out_type=jax.ShapeDtypeStruct((L, NT * D, B), jnp.float32),
    mesh=_mesh,
    compiler_params=pltpu.CompilerParams(
        use_tc_tiling_on_sc=False, needs_layout_passes=False
    ),
    scratch_types=[
        pltpu.VMEM((2, NT, NL, BPW), jnp.int32),
        pltpu.VMEM((2 * ROWS, D), jnp.float32),
        pltpu.VMEM((NL, NT * D, OBP), jnp.float32),
        pltpu.SemaphoreType.DMA,
        pltpu.SemaphoreType.DMA,
    ],
)
def _embed_kernel(w0, w1, w2, w3, i0, i1, i2, i3, out_hbm, idx_v, stage_v, outbuf_v, sem0, sem1):
    tables = (w0, w1, w2, w3)
    idxs = (i0, i1, i2, i3)
    sems = (sem0, sem1)
    wid = lax.axis_index("s") * NC + lax.axis_index("c")
    b0 = wid * BPW
    lane = lax.iota(jnp.int32, 16)
    d_idx = [lane + t * D for t in range(NT)]

    def fetch(c, slot):
        l0 = c * NL
        for t in range(NT):
            pltpu.sync_copy(idxs[t].at[pl.ds(l0, NL), pl.ds(b0, BPW)], idx_v.at[slot, t])
        for l in range(NL):
            for t in range(NT):
                pltpu.async_copy(
                    tables[t].at[idx_v.at[slot, t, l]],
                    stage_v.at[pl.ds(slot * ROWS + (l * NT + t) * BPW, BPW)],
                    sems[slot],
                )

    def drain(slot):
        pltpu.make_async_copy(
            w2.at[pl.ds(0, ROWS)], stage_v.at[pl.ds(slot * ROWS, ROWS)], sems[slot]
        ).wait()

    def process(c, slot):
        base = slot * ROWS

        @plsc.parallel_loop(0, BPW, 1, unroll=UB)
        def tbody(bb):
            bvec = jnp.full((16,), 0, jnp.int32) + bb
            for l in range(NL):
                lvec = jnp.full((16,), l, jnp.int32)
                for t in range(NT):
                    v = stage_v[base + (l * NT + t) * BPW + bb, :]
                    plsc.store_scatter(outbuf_v, [lvec, d_idx[t], bvec], v)
        l0 = c * NL
        for l in range(NL):
            pltpu.sync_copy(
                outbuf_v.at[l, :, pl.ds(0, BPW)],
                out_hbm.at[l0 + l, :, pl.ds(b0, BPW)],
            )

    fetch(0, 0)

    def body(k, carry):
        c0 = 2 * k
        fetch(c0 + 1, 1)
        drain(0)
        process(c0, 0)

        @pl.when(c0 + 2 < NCHUNK)
        def _():
            fetch(c0 + 2, 0)

        drain(1)
        process(c0 + 1, 1)
        return carry

    lax.fori_loop(0, NCHUNK // 2, body, 0)


def kernel(event_name, level, fqid, room_fqid, W_event_name, W_level, W_fqid, W_room_fqid):
    iT = [a.astype(jnp.int32).T for a in (event_name, level, fqid, room_fqid)]
    out = _embed_kernel(W_event_name, W_level, W_fqid, W_room_fqid, *iT)
    return out.transpose(2, 0, 1)


# speedup vs baseline: 2.5686x; 1.0990x over previous
import functools

import jax
import jax.numpy as jnp
from jax import lax
from jax.experimental import pallas as pl
from jax.experimental.pallas import tpu as pltpu
from jax.experimental.pallas import tpu_sc as plsc

B, L, D = 4096, 200, 16
NT = 4
NC, NS = 2, 16
NW = NC * NS
NBG = 8
NLG = NW // NBG
BPW = B // NBG
LPW = L // NLG
ROWS = NT * BPW
UB = 8
OBP = BPW + 1
NSTR = BPW // 128

_mesh = plsc.VectorSubcoreMesh(core_axis_name="c", subcore_axis_name="s")


@functools.partial(
    pl.kernel,
    out_type=jax.ShapeDtypeStruct((L, NT * D, B), jnp.float32),
    mesh=_mesh,
    compiler_params=pltpu.CompilerParams(
        use_tc_tiling_on_sc=False, needs_layout_passes=False
    ),
    scratch_types=[
        pltpu.VMEM((2, NT, BPW), jnp.int32),
        pltpu.VMEM((2 * ROWS, D), jnp.float32),
        pltpu.VMEM((NT * D, OBP), jnp.float32),
        pltpu.SemaphoreType.DMA,
        pltpu.SemaphoreType.DMA,
    ],
)
def _embed_kernel(w0, w1, w2, w3, i0, i1, i2, i3, out_hbm, idx_v, stage_v, outbuf_v, sem0, sem1):
    tables = (w0, w1, w2, w3)
    idxs = (i0, i1, i2, i3)
    sems = (sem0, sem1)
    wid = lax.axis_index("s") * NC + lax.axis_index("c")
    lg = wid // NBG
    b0 = (wid % NBG) * BPW
    l_base = lg * LPW
    lane = lax.iota(jnp.int32, 16)
    d_idx = [lane + t * D for t in range(NT)]

    def fetch(c, slot):
        l = l_base + c
        for t in range(NT):
            pltpu.sync_copy(idxs[t].at[l, pl.ds(b0, BPW)], idx_v.at[slot, t])
        for t in range(NT):
            for j in range(NSTR):
                pltpu.async_copy(
                    tables[t].at[idx_v.at[slot, t, pl.ds(j * 128, 128)]],
                    stage_v.at[pl.ds(slot * ROWS + t * BPW + j * 128, 128)],
                    sems[slot],
                )

    def drain(slot):
        pltpu.make_async_copy(
            w2.at[pl.ds(0, ROWS)], stage_v.at[pl.ds(slot * ROWS, ROWS)], sems[slot]
        ).wait()

    def process(c, slot):
        base = slot * ROWS

        @plsc.parallel_loop(0, BPW, 1, unroll=UB)
        def tbody(bb):
            bvec = jnp.full((16,), 0, jnp.int32) + bb
            for t in range(NT):
                v = stage_v[base + t * BPW + bb, :]
                plsc.store_scatter(outbuf_v, [d_idx[t], bvec], v)

        pltpu.sync_copy(
            outbuf_v.at[:, pl.ds(0, BPW)],
            out_hbm.at[l_base + c, :, pl.ds(b0, BPW)],
        )

    fetch(0, 0)

    def body(k, carry):
        c0 = 2 * k
        fetch(c0 + 1, 1)
        drain(0)
        process(c0, 0)

        @pl.when(c0 + 2 < LPW)
        def _():
            fetch(c0 + 2, 0)

        drain(1)
        process(c0 + 1, 1)
        return carry

    lax.fori_loop(0, LPW // 2, body, 0)


def kernel(event_name, level, fqid, room_fqid, W_event_name, W_level, W_fqid, W_room_fqid):
    iT = [a.astype(jnp.int32).T for a in (event_name, level, fqid, room_fqid)]
    out = _embed_kernel(W_event_name, W_level, W_fqid, W_room_fqid, *iT)
    return out.transpose(2, 0, 1)


# speedup vs baseline: 2.5991x; 1.0119x over previous
import functools

import jax
import jax.numpy as jnp
from jax import lax
from jax.experimental import pallas as pl
from jax.experimental.pallas import tpu as pltpu
from jax.experimental.pallas import tpu_sc as plsc

B, L, D = 4096, 200, 16
NT = 4
NC, NS = 2, 16
NW = NC * NS
NBG = 8
NLG = NW // NBG
BPW = B // NBG
LPW = L // NLG
ROWS = NT * BPW
UB = 8
OBP = BPW + 1
NSTR = BPW // 128

_mesh = plsc.VectorSubcoreMesh(core_axis_name="c", subcore_axis_name="s")


@functools.partial(
    pl.kernel,
    out_type=jax.ShapeDtypeStruct((L, NT * D, B), jnp.float32),
    mesh=_mesh,
    compiler_params=pltpu.CompilerParams(
        use_tc_tiling_on_sc=False, needs_layout_passes=False
    ),
    scratch_types=[
        pltpu.VMEM((2, NT, BPW), jnp.int32),
        pltpu.VMEM((2 * ROWS, D), jnp.float32),
        pltpu.VMEM((NT * D, OBP), jnp.float32),
        pltpu.SemaphoreType.DMA,
        pltpu.SemaphoreType.DMA,
        pltpu.SemaphoreType.DMA,
    ],
)
def _embed_kernel(w0, w1, w2, w3, i0, i1, i2, i3, out_hbm, idx_v, stage_v, outbuf_v, sem0, sem1, sem_w):
    tables = (w0, w1, w2, w3)
    idxs = (i0, i1, i2, i3)
    sems = (sem0, sem1)
    wid = lax.axis_index("s") * NC + lax.axis_index("c")
    lg = wid // NBG
    b0 = (wid % NBG) * BPW
    l_base = lg * LPW
    lane = lax.iota(jnp.int32, 16)
    d_idx = [lane + t * D for t in range(NT)]

    def fetch(c, slot):
        l = l_base + c
        for t in range(NT):
            pltpu.sync_copy(idxs[t].at[l, pl.ds(b0, BPW)], idx_v.at[slot, t])
        for t in range(NT):
            for j in range(NSTR):
                pltpu.async_copy(
                    tables[t].at[idx_v.at[slot, t, pl.ds(j * 128, 128)]],
                    stage_v.at[pl.ds(slot * ROWS + t * BPW + j * 128, 128)],
                    sems[slot],
                )

    def drain(slot):
        pltpu.make_async_copy(
            w2.at[pl.ds(0, ROWS)], stage_v.at[pl.ds(slot * ROWS, ROWS)], sems[slot]
        ).wait()

    def wait_write():
        pltpu.make_async_copy(
            outbuf_v.at[:, pl.ds(0, BPW)],
            out_hbm.at[l_base, :, pl.ds(b0, BPW)],
            sem_w,
        ).wait()

    def process(c, slot):
        base = slot * ROWS
        wait_write()

        @plsc.parallel_loop(0, BPW, 1, unroll=UB)
        def tbody(bb):
            bvec = jnp.full((16,), 0, jnp.int32) + bb
            for t in range(NT):
                v = stage_v[base + t * BPW + bb, :]
                plsc.store_scatter(outbuf_v, [d_idx[t], bvec], v)

        pltpu.async_copy(
            outbuf_v.at[:, pl.ds(0, BPW)],
            out_hbm.at[l_base + c, :, pl.ds(b0, BPW)],
            sem_w,
        )

    fetch(0, 0)
    pltpu.async_copy(
        outbuf_v.at[:, pl.ds(0, BPW)],
        out_hbm.at[l_base, :, pl.ds(b0, BPW)],
        sem_w,
    )

    def body(k, carry):
        c0 = 2 * k
        fetch(c0 + 1, 1)
        drain(0)
        process(c0, 0)

        @pl.when(c0 + 2 < LPW)
        def _():
            fetch(c0 + 2, 0)

        drain(1)
        process(c0 + 1, 1)
        return carry

    lax.fori_loop(0, LPW // 2, body, 0)
    wait_write()


def kernel(event_name, level, fqid, room_fqid, W_event_name, W_level, W_fqid, W_room_fqid):
    iT = [a.astype(jnp.int32).T for a in (event_name, level, fqid, room_fqid)]
    out = _embed_kernel(W_event_name, W_level, W_fqid, W_room_fqid, *iT)
    return out.transpose(2, 0, 1)


# speedup vs baseline: 2.7462x; 1.0566x over previous
import functools

import jax
import jax.numpy as jnp
from jax import lax
from jax.experimental import pallas as pl
from jax.experimental.pallas import tpu as pltpu
from jax.experimental.pallas import tpu_sc as plsc

B, L, D = 4096, 200, 16
NT = 4
NC, NS = 2, 16
NW = NC * NS
NBG = 8
NLG = NW // NBG
BPW = B // NBG
LPW = L // NLG
ROWS = NT * BPW
UB = 8
OBP = BPW + 1
NSTR = BPW // 128

_mesh = plsc.VectorSubcoreMesh(core_axis_name="c", subcore_axis_name="s")


@functools.partial(
    pl.kernel,
    out_type=jax.ShapeDtypeStruct((L, NT * D, B), jnp.float32),
    mesh=_mesh,
    compiler_params=pltpu.CompilerParams(
        use_tc_tiling_on_sc=False, needs_layout_passes=False
    ),
    scratch_types=[
        pltpu.VMEM((2, NT, BPW), jnp.int32),
        pltpu.VMEM((2 * ROWS, D), jnp.float32),
        pltpu.VMEM((NT * D, OBP), jnp.float32),
        pltpu.SemaphoreType.DMA,
        pltpu.SemaphoreType.DMA,
        pltpu.SemaphoreType.DMA,
        pltpu.SemaphoreType.DMA,
        pltpu.SemaphoreType.DMA,
    ],
)
def _embed_kernel(
    w0, w1, w2, w3, i0, i1, i2, i3, out_hbm,
    idx_v, stage_v, outbuf_v, sem0, sem1, semi0, semi1, sem_w,
):
    tables = (w0, w1, w2, w3)
    idxs = (i0, i1, i2, i3)
    sems = (sem0, sem1)
    semis = (semi0, semi1)
    wid = lax.axis_index("s") * NC + lax.axis_index("c")
    lg = wid // NBG
    b0 = (wid % NBG) * BPW
    l_base = lg * LPW
    lane = lax.iota(jnp.int32, 16)
    d_idx = [lane + t * D for t in range(NT)]

    def fetch_idx(c, slot):
        l = l_base + c
        for t in range(NT):
            pltpu.async_copy(
                idxs[t].at[l, pl.ds(b0, BPW)], idx_v.at[slot, t], semis[slot]
            )

    def fire_gathers(c, slot):
        pltpu.make_async_copy(
            i0.at[pl.ds(0, NT), pl.ds(0, BPW)], idx_v.at[slot], semis[slot]
        ).wait()
        for t in range(NT):
            for j in range(NSTR):
                pltpu.async_copy(
                    tables[t].at[idx_v.at[slot, t, pl.ds(j * 128, 128)]],
                    stage_v.at[pl.ds(slot * ROWS + t * BPW + j * 128, 128)],
                    sems[slot],
                )

    def drain(slot):
        pltpu.make_async_copy(
            w2.at[pl.ds(0, ROWS)], stage_v.at[pl.ds(slot * ROWS, ROWS)], sems[slot]
        ).wait()

    def wait_write():
        pltpu.make_async_copy(
            outbuf_v.at[:, pl.ds(0, BPW)],
            out_hbm.at[l_base, :, pl.ds(b0, BPW)],
            sem_w,
        ).wait()

    def process(c, slot):
        base = slot * ROWS
        wait_write()

        @plsc.parallel_loop(0, BPW, 1, unroll=UB)
        def tbody(bb):
            bvec = jnp.full((16,), 0, jnp.int32) + bb
            for t in range(NT):
                v = stage_v[base + t * BPW + bb, :]
                plsc.store_scatter(outbuf_v, [d_idx[t], bvec], v)

        pltpu.async_copy(
            outbuf_v.at[:, pl.ds(0, BPW)],
            out_hbm.at[l_base + c, :, pl.ds(b0, BPW)],
            sem_w,
        )

    fetch_idx(0, 0)
    fire_gathers(0, 0)
    fetch_idx(1, 1)
    pltpu.async_copy(
        outbuf_v.at[:, pl.ds(0, BPW)],
        out_hbm.at[l_base, :, pl.ds(b0, BPW)],
        sem_w,
    )

    def body(k, carry):
        c0 = 2 * k
        fire_gathers(c0 + 1, 1)
        drain(0)

        @pl.when(c0 + 2 < LPW)
        def _():
            fetch_idx(c0 + 2, 0)

        process(c0, 0)

        @pl.when(c0 + 2 < LPW)
        def _():
            fire_gathers(c0 + 2, 0)

        drain(1)

        @pl.when(c0 + 3 < LPW)
        def _():
            fetch_idx(c0 + 3, 1)

        process(c0 + 1, 1)
        return carry

    lax.fori_loop(0, LPW // 2, body, 0)
    wait_write()


def kernel(event_name, level, fqid, room_fqid, W_event_name, W_level, W_fqid, W_room_fqid):
    iT = [a.astype(jnp.int32).T for a in (event_name, level, fqid, room_fqid)]
    out = _embed_kernel(W_event_name, W_level, W_fqid, W_room_fqid, *iT)
    return out.transpose(2, 0, 1)


# speedup vs baseline: 2.7469x; 1.0003x over previous
import functools

import jax
import jax.numpy as jnp
from jax import lax
from jax.experimental import pallas as pl
from jax.experimental.pallas import tpu as pltpu
from jax.experimental.pallas import tpu_sc as plsc

B, L, D = 4096, 200, 16
NT = 4
NC, NS = 2, 16
NW = NC * NS
NBG = 8
NLG = NW // NBG
BPW = B // NBG
LPW = L // NLG
ROWS = NT * BPW
UB = 16
OBP = BPW + 1
NSTR = BPW // 128

_mesh = plsc.VectorSubcoreMesh(core_axis_name="c", subcore_axis_name="s")


@functools.partial(
    pl.kernel,
    out_type=jax.ShapeDtypeStruct((L, NT * D, B), jnp.float32),
    mesh=_mesh,
    compiler_params=pltpu.CompilerParams(
        use_tc_tiling_on_sc=False, needs_layout_passes=False
    ),
    scratch_types=[
        pltpu.VMEM((2, NT, BPW), jnp.int32),
        pltpu.VMEM((2 * ROWS, D), jnp.float32),
        pltpu.VMEM((NT * D, OBP), jnp.float32),
        pltpu.SemaphoreType.DMA,
        pltpu.SemaphoreType.DMA,
        pltpu.SemaphoreType.DMA,
        pltpu.SemaphoreType.DMA,
        pltpu.SemaphoreType.DMA,
    ],
)
def _embed_kernel(
    w0, w1, w2, w3, i0, i1, i2, i3, out_hbm,
    idx_v, stage_v, outbuf_v, sem0, sem1, semi0, semi1, sem_w,
):
    tables = (w0, w1, w2, w3)
    idxs = (i0, i1, i2, i3)
    sems = (sem0, sem1)
    semis = (semi0, semi1)
    wid = lax.axis_index("s") * NC + lax.axis_index("c")
    lg = wid // NBG
    b0 = (wid % NBG) * BPW
    l_base = lg * LPW
    lane = lax.iota(jnp.int32, 16)
    d_idx = [lane + t * D for t in range(NT)]

    def fetch_idx(c, slot):
        l = l_base + c
        for t in range(NT):
            pltpu.async_copy(
                idxs[t].at[l, pl.ds(b0, BPW)], idx_v.at[slot, t], semis[slot]
            )

    def fire_gathers(c, slot):
        pltpu.make_async_copy(
            i0.at[pl.ds(0, NT), pl.ds(0, BPW)], idx_v.at[slot], semis[slot]
        ).wait()
        for t in range(NT):
            for j in range(NSTR):
                pltpu.async_copy(
                    tables[t].at[idx_v.at[slot, t, pl.ds(j * 128, 128)]],
                    stage_v.at[pl.ds(slot * ROWS + t * BPW + j * 128, 128)],
                    sems[slot],
                )

    def drain(slot):
        pltpu.make_async_copy(
            w2.at[pl.ds(0, ROWS)], stage_v.at[pl.ds(slot * ROWS, ROWS)], sems[slot]
        ).wait()

    def wait_write():
        pltpu.make_async_copy(
            outbuf_v.at[:, pl.ds(0, BPW)],
            out_hbm.at[l_base, :, pl.ds(b0, BPW)],
            sem_w,
        ).wait()

    def process(c, slot):
        base = slot * ROWS
        wait_write()

        @plsc.parallel_loop(0, BPW, 1, unroll=UB)
        def tbody(bb):
            bvec = jnp.full((16,), 0, jnp.int32) + bb
            for t in range(NT):
                v = stage_v[base + t * BPW + bb, :]
                plsc.store_scatter(outbuf_v, [d_idx[t], bvec], v)

        pltpu.async_copy(
            outbuf_v.at[:, pl.ds(0, BPW)],
            out_hbm.at[l_base + c, :, pl.ds(b0, BPW)],
            sem_w,
        )

    fetch_idx(0, 0)
    fire_gathers(0, 0)
    fetch_idx(1, 1)
    pltpu.async_copy(
        outbuf_v.at[:, pl.ds(0, BPW)],
        out_hbm.at[l_base, :, pl.ds(b0, BPW)],
        sem_w,
    )

    def body(k, carry):
        c0 = 2 * k
        fire_gathers(c0 + 1, 1)
        drain(0)

        @pl.when(c0 + 2 < LPW)
        def _():
            fetch_idx(c0 + 2, 0)

        process(c0, 0)

        @pl.when(c0 + 2 < LPW)
        def _():
            fire_gathers(c0 + 2, 0)

        drain(1)

        @pl.when(c0 + 3 < LPW)
        def _():
            fetch_idx(c0 + 3, 1)

        process(c0 + 1, 1)
        return carry

    lax.fori_loop(0, LPW // 2, body, 0)
    wait_write()


def kernel(event_name, level, fqid, room_fqid, W_event_name, W_level, W_fqid, W_room_fqid):
    iT = [a.astype(jnp.int32).T for a in (event_name, level, fqid, room_fqid)]
    out = _embed_kernel(W_event_name, W_level, W_fqid, W_room_fqid, *iT)
    return out.transpose(2, 0, 1)
